# baseline (speedup 1.0000x reference)
import functools

import jax
import jax.numpy as jnp
from jax import lax
from jax.experimental import pallas as pl
from jax.experimental.pallas import tpu as pltpu
from jax.experimental.pallas import tpu_sc as plsc

_L = 16
_CH = 512
_G = _CH // 128


def _interp_kernel(n, c, nw):
    n_per_w = n // nw
    n_chunks = n_per_w // _CH
    n_pairs = n_chunks // 2
    mesh = plsc.VectorSubcoreMesh(core_axis_name="c", subcore_axis_name="s")

    @functools.partial(
        pl.kernel,
        mesh=mesh,
        compiler_params=pltpu.CompilerParams(use_tc_tiling_on_sc=False),
        out_type=jax.ShapeDtypeStruct((n, c), jnp.float32),
        scratch_types=[
            pltpu.VMEM((2, 2 * _CH + 8), jnp.float32),
            pltpu.VMEM((2, _G, 128), jnp.int32),
            pltpu.VMEM((2, 2 * _CH), jnp.float32),
            pltpu.VMEM((_CH, c), jnp.float32),
            pltpu.VMEM((_CH, c), jnp.float32),
            pltpu.SemaphoreType.DMA,
            pltpu.SemaphoreType.DMA,
            pltpu.SemaphoreType.DMA,
            pltpu.SemaphoreType.DMA,
        ],
    )
    def body(table_hbm, x_hbm, out_hbm, xp_v, idx_v, w_v,
             rows_a, rows_b, gsem_a, gsem_b, osem_a, osem_b):
        wid = lax.axis_index("s") * 2 + lax.axis_index("c")
        wbase = wid * n_per_w
        rows = (rows_a, rows_b)
        gsem = (gsem_a, gsem_b)
        osem = (osem_a, osem_b)
        lane = lax.iota(jnp.int32, _L)

        def half_vals(p, e):
            va = xp_v[p, pl.ds(e, _L)]
            vb = xp_v[p, pl.ds(e + 1, _L)]
            la = va.astype(jnp.int32)
            lb = vb.astype(jnp.int32)
            w8 = (la.astype(jnp.float32) + 1.0 - va) * (
                lb.astype(jnp.float32) + 1.0 - vb)
            idx8 = jnp.minimum(la, 63) * 64 + jnp.minimum(lb, 63)
            return idx8, w8

        def stage_compute(p, base):
            pltpu.sync_copy(x_hbm.at[pl.ds(2 * base, 2 * _CH)],
                            xp_v.at[p, pl.ds(0, 2 * _CH)])

            def group_body(g, carry):
                for o in range(128 // _L):
                    e = 256 * g + 32 * o
                    idx8a, w8a = half_vals(p, e)
                    idx8b, w8b = half_vals(p, e + 16)
                    dense = jnp.zeros((_L,), jnp.int32)
                    for r in range(8):
                        dense = jnp.where(
                            lane == r,
                            jnp.full((_L,), idx8a[2 * r], dtype=jnp.int32),
                            dense)
                        dense = jnp.where(
                            lane == 8 + r,
                            jnp.full((_L,), idx8b[2 * r], dtype=jnp.int32),
                            dense)
                    idx_v[p, g, pl.ds(o * _L, _L)] = dense
                    w_v[p, pl.ds(e, _L)] = w8a
                    w_v[p, pl.ds(e + 16, _L)] = w8b
                return carry

            lax.fori_loop(0, _G, group_body, 0)

        def fire_gathers(p):
            for g in range(_G):
                pltpu.async_copy(table_hbm.at[idx_v.at[p, g]],
                                 rows[p].at[pl.ds(g * 128, 128)], gsem[p])

        def wait_gathers(p):
            for g in range(_G):
                pltpu.make_async_copy(table_hbm.at[idx_v.at[p, g]],
                                      rows[p].at[pl.ds(g * 128, 128)],
                                      gsem[p]).wait()

        def scale(p):
            def scale_body(jb, carry):
                wa = w_v[p, pl.ds(32 * jb, _L)]
                wb = w_v[p, pl.ds(32 * jb + 16, _L)]
                for r in range(_L):
                    j = jb * _L + r
                    src = wa if r < 8 else wb
                    wj = jnp.full((_L,), src[2 * (r % 8)], dtype=jnp.float32)
                    for k in range(c // _L):
                        sl = pl.ds(k * _L, _L)
                        rows[p][j, sl] = rows[p][j, sl] * wj
                return carry

            lax.fori_loop(0, _CH // _L, scale_body, 0)

        def fire_out(p, base):
            pltpu.async_copy(rows[p], out_hbm.at[pl.ds(base, _CH)], osem[p])

        def wait_out(p):
            pltpu.make_async_copy(rows[p], out_hbm.at[pl.ds(0, _CH)],
                                  osem[p]).wait()

        stage_compute(0, wbase)
        fire_gathers(0)

        def pair_body(k, carry):
            base_a = wbase + (2 * k) * _CH

            stage_compute(1, base_a + _CH)

            @pl.when(k > 0)
            def _():
                wait_out(1)
            fire_gathers(1)

            wait_gathers(0)
            scale(0)
            fire_out(0, base_a)

            @pl.when(k < n_pairs - 1)
            def _():
                stage_compute(0, base_a + 2 * _CH)
                wait_out(0)
                fire_gathers(0)

            wait_gathers(1)
            scale(1)
            fire_out(1, base_a + _CH)
            return carry

        lax.fori_loop(0, n_pairs, pair_body, 0)
        wait_out(0)
        wait_out(1)

    return body


def kernel(image, x):
    h, w, c = image.shape
    n = x.shape[0]
    table = image.reshape(h * w, c)
    info = plsc.get_sparse_core_info()
    nw = info.num_cores * info.num_subcores
    assert n % (nw * 2 * _CH) == 0
    return _interp_kernel(n, c, nw)(table, x.reshape(2 * n))


# speedup vs baseline: 1.6100x; 1.6100x over previous
import functools

import jax
import jax.numpy as jnp
from jax import lax
from jax.experimental import pallas as pl
from jax.experimental.pallas import tpu as pltpu
from jax.experimental.pallas import tpu_sc as plsc

_L = 16
_CH = 256
_G = _CH // 128


def _interp_kernel(n, c, nw):
    n_per_w = n // nw
    n_chunks = n_per_w // _CH
    n_pairs = n_chunks // 2
    mesh = plsc.VectorSubcoreMesh(core_axis_name="c", subcore_axis_name="s")

    @functools.partial(
        pl.kernel,
        mesh=mesh,
        compiler_params=pltpu.CompilerParams(use_tc_tiling_on_sc=False),
        out_type=jax.ShapeDtypeStruct((n * c // 128, 128), jnp.float32),
        scratch_types=[
            pltpu.VMEM((2, 2 * _CH), jnp.float32),
            pltpu.VMEM((2, _G, 128), jnp.int32),
            pltpu.VMEM((2, _CH), jnp.float32),
            pltpu.VMEM((_CH, c), jnp.float32),
            pltpu.VMEM((_CH, c), jnp.float32),
            pltpu.VMEM((_CH * c // 128, 128), jnp.float32),
            pltpu.VMEM((_CH * c // 128, 128), jnp.float32),
            pltpu.SemaphoreType.DMA,
            pltpu.SemaphoreType.DMA,
            pltpu.SemaphoreType.DMA,
            pltpu.SemaphoreType.DMA,
        ],
    )
    def body(table_hbm, x_hbm, out_hbm, xp_v, idx_v, w_v,
             rows_a, rows_b, stage_a, stage_b,
             gsem_a, gsem_b, osem_a, osem_b):
        wid = lax.axis_index("s") * 2 + lax.axis_index("c")
        wbase = wid * n_per_w
        rows = (rows_a, rows_b)
        stage = (stage_a, stage_b)
        gsem = (gsem_a, gsem_b)
        osem = (osem_a, osem_b)
        out_rows = _CH * c // 128

        def stage_compute(p, base):
            pltpu.sync_copy(x_hbm.at[pl.ds(2 * base, 2 * _CH)],
                            xp_v.at[p])
            for g in range(_G):
                for o in range(128 // _L):
                    x0 = xp_v[p, pl.ds(256 * g + _L * o, _L)]
                    x1 = xp_v[p, pl.ds(256 * g + 128 + _L * o, _L)]
                    l0 = x0.astype(jnp.int32)
                    l1 = x1.astype(jnp.int32)
                    w_v[p, pl.ds(128 * g + _L * o, _L)] = (
                        l0.astype(jnp.float32) + 1.0 - x0) * (
                        l1.astype(jnp.float32) + 1.0 - x1)
                    idx_v[p, g, pl.ds(_L * o, _L)] = (
                        jnp.minimum(l0, 63) * 64 + jnp.minimum(l1, 63))

        def fire_gathers(p):
            for g in range(_G):
                pltpu.async_copy(table_hbm.at[idx_v.at[p, g]],
                                 rows[p].at[pl.ds(g * 128, 128)], gsem[p])

        def wait_gathers(p):
            for g in range(_G):
                pltpu.make_async_copy(table_hbm.at[idx_v.at[p, g]],
                                      rows[p].at[pl.ds(g * 128, 128)],
                                      gsem[p]).wait()

        def scale(p):
            def scale_body(jb, carry):
                w16 = w_v[p, pl.ds(jb * _L, _L)]
                for r in range(_L):
                    j = jb * _L + r
                    wj = jnp.full((_L,), w16[r], dtype=jnp.float32)
                    half = (j % 2) * c
                    for k in range(c // _L):
                        stage[p][j // 2, pl.ds(half + k * _L, _L)] = (
                            rows[p][j, pl.ds(k * _L, _L)] * wj)
                return carry

            lax.fori_loop(0, _CH // _L, scale_body, 0)

        def fire_out(p, base):
            pltpu.async_copy(stage[p],
                             out_hbm.at[pl.ds(base * c // 128, out_rows)],
                             osem[p])

        def wait_out(p):
            pltpu.make_async_copy(stage[p],
                                  out_hbm.at[pl.ds(0, out_rows)],
                                  osem[p]).wait()

        stage_compute(0, wbase)
        fire_gathers(0)

        def pair_body(k, carry):
            base_a = wbase + (2 * k) * _CH

            stage_compute(1, base_a + _CH)

            @pl.when(k > 0)
            def _():
                wait_out(1)
            fire_gathers(1)

            wait_gathers(0)
            scale(0)
            fire_out(0, base_a)

            @pl.when(k < n_pairs - 1)
            def _():
                stage_compute(0, base_a + 2 * _CH)
                wait_out(0)
                fire_gathers(0)

            wait_gathers(1)
            scale(1)
            fire_out(1, base_a + _CH)
            return carry

        lax.fori_loop(0, n_pairs, pair_body, 0)
        wait_out(0)
        wait_out(1)

    return body


def kernel(image, x):
    h, w, c = image.shape
    n = x.shape[0]
    table = image.reshape(h * w, c)
    info = plsc.get_sparse_core_info()
    nw = info.num_cores * info.num_subcores
    assert n % (nw * 2 * _CH) == 0 and n % 128 == 0 and (_CH * c) % 128 == 0
    xv = x.reshape(n // 128, 128, 2).transpose(0, 2, 1).reshape(2 * n)
    out128 = _interp_kernel(n, c, nw)(table, xv)
    return out128.reshape(n, c)
